# baseline (speedup 1.0000x reference)
import math

import jax
import jax.numpy as jnp
import numpy as np
from jax.experimental import pallas as pl
from jax.experimental.pallas import tpu as pltpu

D, H, NQ, NT, NL, NR, MAX_AGENT = 1024, 256, 8192, 16, 8, 8, 2
BM = 512
NP = NL + NL * NL
NPP = 80

_LN2 = 0.6931471805599453
_LOG3 = math.log(3.0)

_PHI_XS = np.linspace(-1.0, 1.0, 4001)
_PHI = tuple(
    float(c) for c in np.polyfit(
        _PHI_XS,
        np.array([math.lgamma(2.0 / (1.0 + math.exp(-x)) + 1.0)
                  for x in _PHI_XS]),
        10))


def _phi(d):
    t = jnp.clip(d, -1.0, 1.0)
    acc = jnp.full_like(t, _PHI[0])
    for c in _PHI[1:]:
        acc = acc * t + c
    return acc

_BF = jnp.bfloat16
_HP = jax.lax.Precision.HIGHEST


def _l2n(x):
    n = jnp.sqrt(jnp.sum(x * x, axis=1, keepdims=True))
    return x * (1.0 / jnp.maximum(n, 1e-12))


def _dot_t(a, b):
    return jax.lax.dot_general(a.astype(_BF), b.astype(_BF),
                               (((1,), (1,)), ((), ())),
                               preferred_element_type=jnp.float32)


def _softmax(x):
    m = jnp.max(x, axis=1, keepdims=True)
    e = jnp.exp(x - m)
    return e / jnp.sum(e, axis=1, keepdims=True)


def _first_true_idx(mask, iota_f, width):
    cand = jnp.where(mask, iota_f, float(width))
    idx = jnp.min(cand, axis=1, keepdims=True)
    return jnp.where(idx == float(width), 0.0, idx)


def _split3(x):
    hi = x.astype(_BF)
    r = x - hi.astype(jnp.float32)
    mid = r.astype(_BF)
    lo = (r - mid.astype(jnp.float32)).astype(_BF)
    return hi, mid, lo


def _iota_f(shape, dim):
    return jax.lax.broadcasted_iota(jnp.int32, shape, dim).astype(jnp.float32)


def _onehot3(sel_i32, stride):
    i3 = jax.lax.broadcasted_iota(jnp.int32, (BM, 3 * stride), 1)
    return (jax.lax.rem(i3, stride) == sel_i32).astype(_BF)


def _router_kernel(q_ref, wq_ref, wqt_a_ref, wqt_b_ref, wqtl_a_ref,
                   wqtl_b_ref, wqtl_c_ref, wt_ref, wl_ref, wr_ref,
                   tasks_ref, lr_ref, bias_ref, rnd_ref,
                   ts_ref, sel_ref, lgp_ref, rsel_ref, rlp_ref,
                   wcat_bf, te_tab, le_tab, re_tab, tq3, ptab3, pairs_bf):
    @pl.when(pl.program_id(0) == 0)
    def _prologue():
        wcat_bf[:, 0:H] = wq_ref[...].astype(_BF)
        wcat_bf[:, H:2 * H] = wqt_a_ref[...].astype(_BF)
        wcat_bf[:, 2 * H:3 * H] = wqtl_a_ref[...].astype(_BF)
        tasks_bf = tasks_ref[...].astype(_BF)
        te_tab[...] = _l2n(
            jnp.dot(tasks_bf, wt_ref[...].astype(_BF),
                    preferred_element_type=jnp.float32) + bias_ref[3:4, :])
        xqt = jnp.dot(tasks_bf, wqt_b_ref[...].astype(_BF),
                      preferred_element_type=jnp.float32)
        xqtl = jnp.dot(tasks_bf, wqtl_b_ref[...].astype(_BF),
                       preferred_element_type=jnp.float32)
        h, m, l = _split3(jnp.concatenate([xqt, xqtl], axis=1))
        tq3[0:NT, :] = h
        tq3[NT:2 * NT, :] = m
        tq3[2 * NT:3 * NT, :] = l
        lr_bf = lr_ref[...].astype(_BF)
        cl = jnp.dot(lr_bf, wl_ref[...].astype(_BF),
                     preferred_element_type=jnp.float32)
        le_tab[0:NL, :] = _l2n(cl[0:NL, :] + bias_ref[4:5, :])
        le_tab[NL:NL + 1, :] = bias_ref[7:8, :]
        le_tab[NL + 1:16, :] = jnp.zeros((16 - NL - 1, H), jnp.float32)
        cr = jnp.dot(lr_bf, wr_ref[...].astype(_BF),
                     preferred_element_type=jnp.float32)
        re_tab[...] = _l2n(cr[NL:NL + NR, :] + bias_ref[5:6, :])
        w3_bf = wqtl_c_ref[...].astype(_BF)
        t1 = jnp.dot(lr_bf, w3_bf, preferred_element_type=jnp.float32)[0:NL, :]
        lfr = lr_ref[0:NL, :].astype(_BF).astype(jnp.float32)
        for a in range(NL):
            pairs_bf[NL * a:NL * a + NL, :] = (
                (lfr[a:a + 1, :] + lfr).astype(_BF))
        pp = jnp.dot(pairs_bf[...], w3_bf, preferred_element_type=jnp.float32)
        h, m, l = _split3(t1)
        ptab3[0:NL, :] = h
        ptab3[NPP:NPP + NL, :] = m
        ptab3[2 * NPP:2 * NPP + NL, :] = l
        h, m, l = _split3(pp)
        ptab3[NL:NP, :] = h
        ptab3[NPP + NL:NPP + NP, :] = m
        ptab3[2 * NPP + NL:2 * NPP + NP, :] = l
        zpad = jnp.zeros((NPP - NP, H), _BF)
        ptab3[NP:NPP, :] = zpad
        ptab3[NPP + NP:2 * NPP, :] = zpad
        ptab3[2 * NPP + NP:3 * NPP, :] = zpad

    q_bf = q_ref[...].astype(_BF)
    p1 = jnp.dot(q_bf, wcat_bf[:, 0:H], preferred_element_type=jnp.float32)
    p2 = jnp.dot(q_bf, wcat_bf[:, H:2 * H], preferred_element_type=jnp.float32)
    p3 = jnp.dot(q_bf, wcat_bf[:, 2 * H:3 * H],
                 preferred_element_type=jnp.float32)

    qe = _l2n(p1 + bias_ref[0:1, :])
    ts = _dot_t(qe, te_tab[...])
    ts_ref[...] = ts
    iota_t = _iota_f((BM, NT), 1)
    mx = jnp.max(ts, axis=1, keepdims=True)
    sel_task = _first_true_idx(ts == mx, iota_t, NT)

    g = jnp.dot(_onehot3(sel_task.astype(jnp.int32), NT), tq3[...],
                preferred_element_type=jnp.float32)

    qt = _l2n(p2 + g[:, 0:H] + bias_ref[1:2, :])
    logits_ext = _dot_t(qt, le_tab[...])
    dlogit = logits_ext[:, NL:NL + 1] + bias_ref[6:7, 0:1]
    mask2 = (dlogit >= _LOG3).astype(jnp.float32)

    scores = _softmax(logits_ext[:, 0:NL])
    iota_l = _iota_f((BM, NL), 1)
    tri = (jax.lax.broadcasted_iota(jnp.int32, (NL, NL), 0)
           <= jax.lax.broadcasted_iota(jnp.int32, (NL, NL), 1)).astype(jnp.float32)
    sc = jnp.dot(scores, tri, preferred_element_type=jnp.float32,
                 precision=_HP)

    idx1 = _first_true_idx(sc > rnd_ref[:, 0:1], iota_l, NL)
    idx2 = _first_true_idx(sc > rnd_ref[:, 1:2], iota_l, NL)
    selected = ((iota_l == idx1).astype(jnp.float32)
                + mask2 * (iota_l == idx2).astype(jnp.float32))
    sel_ref[...] = selected

    pair_ln2 = mask2 * jnp.where(idx1 == idx2, _LN2, 0.0)
    lgp_ref[...] = (_phi(dlogit) - pair_ln2
                    + jnp.sum(selected * jnp.log(scores), axis=1, keepdims=True))

    sel72 = jnp.where(mask2 > 0.0, 8.0 + 8.0 * idx1 + idx2, idx1)
    gp = jnp.dot(_onehot3(sel72.astype(jnp.int32), NPP), ptab3[...],
                 preferred_element_type=jnp.float32)
    qtl = _l2n(p3 + g[:, H:2 * H] + gp + bias_ref[2:3, :])
    rlogits = _dot_t(qtl, re_tab[...])
    rscores = _softmax(rlogits)
    rsc = jnp.dot(rscores, tri, preferred_element_type=jnp.float32,
                  precision=_HP)
    rnd2 = rnd_ref[:, 2:3]
    ridx = _first_true_idx(rsc > rnd2, iota_l, NR)
    rsel_ref[...] = ridx.astype(jnp.int32)
    oh_r = (iota_l == ridx).astype(jnp.float32)
    rlp_ref[...] = jnp.log(jnp.sum(oh_r * rscores, axis=1, keepdims=True))


def kernel(queries, tasks, llms, reasonings, Wq, bq, Wt, bt, Wqt, bqt,
           Wl, bl, Wd, bd, Wqtl, bqtl, Wr, br):
    f32 = jnp.float32
    lr = jnp.concatenate([llms, reasonings], axis=0)
    bias = jnp.stack([bq, bqt, bqtl, bt, bl, br,
                      jnp.broadcast_to(bd, (H,)), Wd[:, 0]], axis=0)

    rkey = jax.random.key(42)
    rnd_cols = [jax.random.uniform(jax.random.fold_in(rkey, i), (NQ, 1))
                for i in range(1, MAX_AGENT + 1)]
    rnd_cols.append(jax.random.uniform(jax.random.fold_in(rkey, 999), (NQ, 1)))
    rnd_cols.append(jnp.zeros((NQ, 1), f32))
    rnd = jnp.concatenate(rnd_cols, axis=1)

    grid = (NQ // BM,)
    row_spec = lambda w: pl.BlockSpec((BM, w), lambda i: (i, 0))
    rep_spec = lambda r, w: pl.BlockSpec((r, w), lambda i: (0, 0))
    w_spec = lambda blk: pl.BlockSpec((D, H), lambda i, _b=blk: (_b, 0))

    out = pl.pallas_call(
        _router_kernel,
        grid=grid,
        in_specs=[
            row_spec(D),
            rep_spec(D, H),
            w_spec(0),
            w_spec(1),
            w_spec(0),
            w_spec(1),
            w_spec(2),
            rep_spec(D, H),
            rep_spec(D, H),
            rep_spec(D, H),
            rep_spec(NT, D),
            rep_spec(16, D),
            rep_spec(8, H),
            row_spec(4),
        ],
        out_specs=[
            row_spec(NT), row_spec(NL), row_spec(1), row_spec(1), row_spec(1),
        ],
        out_shape=[
            jax.ShapeDtypeStruct((NQ, NT), f32),
            jax.ShapeDtypeStruct((NQ, NL), f32),
            jax.ShapeDtypeStruct((NQ, 1), f32),
            jax.ShapeDtypeStruct((NQ, 1), jnp.int32),
            jax.ShapeDtypeStruct((NQ, 1), f32),
        ],
        scratch_shapes=[
            pltpu.VMEM((D, 3 * H), _BF),
            pltpu.VMEM((NT, H), f32),
            pltpu.VMEM((16, H), f32),
            pltpu.VMEM((NR, H), f32),
            pltpu.VMEM((3 * NT, 2 * H), _BF),
            pltpu.VMEM((3 * NPP, H), _BF),
            pltpu.VMEM((NL * NL, D), _BF),
        ],
    )(queries, Wq, Wqt, Wqt, Wqtl, Wqtl, Wqtl, Wt, Wl, Wr,
      tasks, lr, bias, rnd)

    task_scores, selected_llm, llm_log_probs, r_sel, r_log_probs = out
    return (task_scores, selected_llm, llm_log_probs,
            r_sel.reshape(NQ), r_log_probs)


# speedup vs baseline: 1.1888x; 1.1888x over previous
import math

import jax
import jax.numpy as jnp
import numpy as np
from jax.experimental import pallas as pl
from jax.experimental.pallas import tpu as pltpu

D, H, NQ, NT, NL, NR, MAX_AGENT = 1024, 256, 8192, 16, 8, 8, 2
BM = 512
NP = NL + NL * NL
NPP = 80

_LN2 = 0.6931471805599453
_LOG3 = math.log(3.0)

_PHI_XS = np.linspace(-1.0, 1.0, 4001)
_PHI = tuple(
    float(c) for c in np.polyfit(
        _PHI_XS,
        np.array([math.lgamma(2.0 / (1.0 + math.exp(-x)) + 1.0)
                  for x in _PHI_XS]),
        10))


def _phi(d):
    t = jnp.clip(d, -1.0, 1.0)
    acc = jnp.full_like(t, _PHI[0])
    for c in _PHI[1:]:
        acc = acc * t + c
    return acc

_BF = jnp.bfloat16
_HP = jax.lax.Precision.HIGHEST


def _l2n(x):
    n = jnp.sqrt(jnp.sum(x * x, axis=1, keepdims=True))
    return x * (1.0 / jnp.maximum(n, 1e-12))


def _dot_t(a, b):
    return jax.lax.dot_general(a.astype(_BF), b.astype(_BF),
                               (((1,), (1,)), ((), ())),
                               preferred_element_type=jnp.float32)


def _softmax(x):
    m = jnp.max(x, axis=1, keepdims=True)
    e = jnp.exp(x - m)
    return e / jnp.sum(e, axis=1, keepdims=True)


def _first_true_idx(mask, iota_f, width):
    cand = jnp.where(mask, iota_f, float(width))
    idx = jnp.min(cand, axis=1, keepdims=True)
    return jnp.where(idx == float(width), 0.0, idx)


def _split3(x):
    hi = x.astype(_BF)
    r = x - hi.astype(jnp.float32)
    mid = r.astype(_BF)
    lo = (r - mid.astype(jnp.float32)).astype(_BF)
    return hi, mid, lo


def _iota_f(shape, dim):
    return jax.lax.broadcasted_iota(jnp.int32, shape, dim).astype(jnp.float32)


def _onehot3(sel_i32, stride):
    i3 = jax.lax.broadcasted_iota(jnp.int32, (BM, 3 * stride), 1)
    return (jax.lax.rem(i3, stride) == sel_i32).astype(_BF)


def _router_kernel(q_ref, wq_ref, wqt_a_ref, wqt_b_ref, wqtl_a_ref,
                   wqtl_b_ref, wqtl_c_ref, wt_ref, wl_ref, wr_ref,
                   tasks_ref, lr_ref, bias_ref, rnd_ref,
                   ts_ref, sel_ref, lgp_ref, rsel_ref, rlp_ref,
                   wcat_bf, te_tab, le_tab, re_tab, tq3, ptab3, pairs_bf):
    @pl.when(pl.program_id(0) == 0)
    def _prologue():
        wcat_bf[:, 0:H] = wq_ref[...].astype(_BF)
        wcat_bf[:, H:2 * H] = wqt_a_ref[...].astype(_BF)
        wcat_bf[:, 2 * H:3 * H] = wqtl_a_ref[...].astype(_BF)
        tasks_bf = tasks_ref[...].astype(_BF)
        te_tab[...] = _l2n(
            jnp.dot(tasks_bf, wt_ref[...].astype(_BF),
                    preferred_element_type=jnp.float32) + bias_ref[3:4, :])
        xqt = jnp.dot(tasks_bf, wqt_b_ref[...].astype(_BF),
                      preferred_element_type=jnp.float32)
        xqtl = jnp.dot(tasks_bf, wqtl_b_ref[...].astype(_BF),
                       preferred_element_type=jnp.float32)
        h, m, l = _split3(jnp.concatenate([xqt, xqtl], axis=1))
        tq3[0:NT, :] = h
        tq3[NT:2 * NT, :] = m
        tq3[2 * NT:3 * NT, :] = l
        lr_bf = lr_ref[...].astype(_BF)
        cl = jnp.dot(lr_bf, wl_ref[...].astype(_BF),
                     preferred_element_type=jnp.float32)
        le_tab[0:NL, :] = _l2n(cl[0:NL, :] + bias_ref[4:5, :])
        le_tab[NL:NL + 1, :] = bias_ref[7:8, :]
        le_tab[NL + 1:16, :] = jnp.zeros((16 - NL - 1, H), jnp.float32)
        cr = jnp.dot(lr_bf, wr_ref[...].astype(_BF),
                     preferred_element_type=jnp.float32)
        re_tab[...] = _l2n(cr[NL:NL + NR, :] + bias_ref[5:6, :])
        w3_bf = wqtl_c_ref[...].astype(_BF)
        t1 = jnp.dot(lr_bf, w3_bf, preferred_element_type=jnp.float32)[0:NL, :]
        lfr = lr_ref[0:NL, :].astype(_BF).astype(jnp.float32)
        for a in range(NL):
            pairs_bf[NL * a:NL * a + NL, :] = (
                (lfr[a:a + 1, :] + lfr).astype(_BF))
        pp = jnp.dot(pairs_bf[...], w3_bf, preferred_element_type=jnp.float32)
        h, m, l = _split3(t1)
        ptab3[0:NL, :] = h
        ptab3[NPP:NPP + NL, :] = m
        ptab3[2 * NPP:2 * NPP + NL, :] = l
        h, m, l = _split3(pp)
        ptab3[NL:NP, :] = h
        ptab3[NPP + NL:NPP + NP, :] = m
        ptab3[2 * NPP + NL:2 * NPP + NP, :] = l
        zpad = jnp.zeros((NPP - NP, H), _BF)
        ptab3[NP:NPP, :] = zpad
        ptab3[NPP + NP:2 * NPP, :] = zpad
        ptab3[2 * NPP + NP:3 * NPP, :] = zpad

    q_bf = q_ref[...].astype(_BF)
    p = jnp.dot(q_bf, wcat_bf[...], preferred_element_type=jnp.float32)

    qe = _l2n(p[:, 0:H] + bias_ref[0:1, :])
    ts = _dot_t(qe, te_tab[...])
    ts_ref[...] = ts
    iota_t = _iota_f((BM, NT), 1)
    mx = jnp.max(ts, axis=1, keepdims=True)
    sel_task = _first_true_idx(ts == mx, iota_t, NT)

    g = jnp.dot(_onehot3(sel_task.astype(jnp.int32), NT), tq3[...],
                preferred_element_type=jnp.float32)

    qt = _l2n(p[:, H:2 * H] + g[:, 0:H] + bias_ref[1:2, :])
    logits_ext = _dot_t(qt, le_tab[...])
    dlogit = logits_ext[:, NL:NL + 1] + bias_ref[6:7, 0:1]
    mask2 = (dlogit >= _LOG3).astype(jnp.float32)

    scores = _softmax(logits_ext[:, 0:NL])
    iota_l = _iota_f((BM, NL), 1)
    tri = (jax.lax.broadcasted_iota(jnp.int32, (NL, NL), 0)
           <= jax.lax.broadcasted_iota(jnp.int32, (NL, NL), 1)).astype(jnp.float32)
    sc = jnp.dot(scores, tri, preferred_element_type=jnp.float32,
                 precision=_HP)

    idx1 = _first_true_idx(sc > rnd_ref[:, 0:1], iota_l, NL)
    idx2 = _first_true_idx(sc > rnd_ref[:, 1:2], iota_l, NL)
    selected = ((iota_l == idx1).astype(jnp.float32)
                + mask2 * (iota_l == idx2).astype(jnp.float32))
    sel_ref[...] = selected

    pair_ln2 = mask2 * jnp.where(idx1 == idx2, _LN2, 0.0)
    lgp_ref[...] = (_phi(dlogit) - pair_ln2
                    + jnp.sum(selected * jnp.log(scores), axis=1, keepdims=True))

    sel72 = jnp.where(mask2 > 0.0, 8.0 + 8.0 * idx1 + idx2, idx1)
    gp = jnp.dot(_onehot3(sel72.astype(jnp.int32), NPP), ptab3[...],
                 preferred_element_type=jnp.float32)
    qtl = _l2n(p[:, 2 * H:3 * H] + g[:, H:2 * H] + gp + bias_ref[2:3, :])
    rlogits = _dot_t(qtl, re_tab[...])
    rscores = _softmax(rlogits)
    rsc = jnp.dot(rscores, tri, preferred_element_type=jnp.float32,
                  precision=_HP)
    rnd2 = rnd_ref[:, 2:3]
    ridx = _first_true_idx(rsc > rnd2, iota_l, NR)
    rsel_ref[...] = ridx.astype(jnp.int32)
    oh_r = (iota_l == ridx).astype(jnp.float32)
    rlp_ref[...] = jnp.log(jnp.sum(oh_r * rscores, axis=1, keepdims=True))


def kernel(queries, tasks, llms, reasonings, Wq, bq, Wt, bt, Wqt, bqt,
           Wl, bl, Wd, bd, Wqtl, bqtl, Wr, br):
    f32 = jnp.float32
    lr = jnp.concatenate([llms, reasonings], axis=0)
    bias = jnp.stack([bq, bqt, bqtl, bt, bl, br,
                      jnp.broadcast_to(bd, (H,)), Wd[:, 0]], axis=0)

    rkey = jax.random.key(42)
    rnd_cols = [jax.random.uniform(jax.random.fold_in(rkey, i), (NQ, 1))
                for i in range(1, MAX_AGENT + 1)]
    rnd_cols.append(jax.random.uniform(jax.random.fold_in(rkey, 999), (NQ, 1)))
    rnd_cols.append(jnp.zeros((NQ, 1), f32))
    rnd = jnp.concatenate(rnd_cols, axis=1)

    grid = (NQ // BM,)
    row_spec = lambda w: pl.BlockSpec((BM, w), lambda i: (i, 0))
    rep_spec = lambda r, w: pl.BlockSpec((r, w), lambda i: (0, 0))
    w_spec = lambda blk: pl.BlockSpec((D, H), lambda i, _b=blk: (_b, 0))

    out = pl.pallas_call(
        _router_kernel,
        grid=grid,
        in_specs=[
            row_spec(D),
            rep_spec(D, H),
            w_spec(0),
            w_spec(1),
            w_spec(0),
            w_spec(1),
            w_spec(2),
            rep_spec(D, H),
            rep_spec(D, H),
            rep_spec(D, H),
            rep_spec(NT, D),
            rep_spec(16, D),
            rep_spec(8, H),
            row_spec(4),
        ],
        out_specs=[
            row_spec(NT), row_spec(NL), row_spec(1), row_spec(1), row_spec(1),
        ],
        out_shape=[
            jax.ShapeDtypeStruct((NQ, NT), f32),
            jax.ShapeDtypeStruct((NQ, NL), f32),
            jax.ShapeDtypeStruct((NQ, 1), f32),
            jax.ShapeDtypeStruct((NQ, 1), jnp.int32),
            jax.ShapeDtypeStruct((NQ, 1), f32),
        ],
        scratch_shapes=[
            pltpu.VMEM((D, 3 * H), _BF),
            pltpu.VMEM((NT, H), f32),
            pltpu.VMEM((16, H), f32),
            pltpu.VMEM((NR, H), f32),
            pltpu.VMEM((3 * NT, 2 * H), _BF),
            pltpu.VMEM((3 * NPP, H), _BF),
            pltpu.VMEM((NL * NL, D), _BF),
        ],
    )(queries, Wq, Wqt, Wqt, Wqtl, Wqtl, Wqtl, Wt, Wl, Wr,
      tasks, lr, bias, rnd)

    task_scores, selected_llm, llm_log_probs, r_sel, r_log_probs = out
    return (task_scores, selected_llm, llm_log_probs,
            r_sel.reshape(NQ), r_log_probs)
